# baseline (speedup 1.0000x reference)
import functools

import jax
import jax.numpy as jnp
from jax import lax
from jax.experimental import pallas as pl
from jax.experimental.pallas import tpu as pltpu
from jax.experimental.pallas import tpu_sc as plsc

_B = 16384
_D = 64
_NC = 2
_NS = 16
_NW = _NC * _NS
_BW = _B // _NW
_CH = 256
_NCHUNK = _BW // _CH
_L = 16


def _make_kernel():
    mesh = plsc.VectorSubcoreMesh(core_axis_name="c", subcore_axis_name="s")

    @functools.partial(
        pl.kernel,
        mesh=mesh,
        compiler_params=pltpu.CompilerParams(needs_layout_passes=False),
        out_type=[
            jax.ShapeDtypeStruct((_B,), jnp.float32),
            jax.ShapeDtypeStruct((_B,), jnp.float32),
        ],
        scratch_types=[
            pltpu.VMEM((_CH,), jnp.int32),
            pltpu.VMEM((_CH,), jnp.int32),
            pltpu.VMEM((_CH,), jnp.int32),
            pltpu.VMEM((_CH,), jnp.int32),
            pltpu.VMEM((_CH,), jnp.int32),
            pltpu.VMEM((_CH,), jnp.int32),
            pltpu.VMEM((_CH, 2 * _D), jnp.float32),
            pltpu.VMEM((_CH, 2 * _D), jnp.float32),
            pltpu.VMEM((_CH, 2 * _D), jnp.float32),
            pltpu.VMEM((_BW,), jnp.float32),
            pltpu.SemaphoreType.DMA,
            pltpu.SemaphoreType.DMA,
            pltpu.SemaphoreType.DMA,
        ],
    )
    def trans_e(ph, pr, pt, nh, nr, nt, ent2, rel2, pos_out, neg_out,
                idx_h, idx_r, idx_t, par_h, par_r, par_t,
                hrows, rrows, trows, obuf, sem_h, sem_r, sem_t):
        wid = lax.axis_index("s") * _NC + lax.axis_index("c")
        base = wid * _BW
        lanes = lax.iota(jnp.int32, _L)

        def stage_idx(src_hbm, off, idx_v, par_v):
            pltpu.sync_copy(src_hbm.at[pl.ds(off, _CH)], idx_v)

            def body(k, carry):
                raw = idx_v[pl.ds(k * _L, _L)]
                par_v[pl.ds(k * _L, _L)] = lax.shift_left(
                    lax.bitwise_and(raw, 1), 6)
                idx_v[pl.ds(k * _L, _L)] = lax.shift_right_logical(raw, 1)
                return carry

            lax.fori_loop(0, _CH // _L, body, 0)

        def one_side(h_hbm, r_hbm, t_hbm, out_hbm):
            for chunk in range(_NCHUNK):
                off = base + chunk * _CH
                stage_idx(h_hbm, off, idx_h, par_h)
                stage_idx(r_hbm, off, idx_r, par_r)
                stage_idx(t_hbm, off, idx_t, par_t)
                chd = pltpu.async_copy(ent2.at[idx_h], hrows, sem_h)
                crd = pltpu.async_copy(rel2.at[idx_r], rrows, sem_r)
                ctd = pltpu.async_copy(ent2.at[idx_t], trows, sem_t)
                chd.wait()
                crd.wait()
                ctd.wait()

                def group(g, carry):
                    vec = jnp.zeros((_L,), jnp.float32)
                    pv_h = par_h[pl.ds(g * _L, _L)]
                    pv_r = par_r[pl.ds(g * _L, _L)]
                    pv_t = par_t[pl.ds(g * _L, _L)]
                    for j in range(_L):
                        i = g * _L + j
                        oh = pv_h[j]
                        orr = pv_r[j]
                        ot = pv_t[j]
                        acc = jnp.zeros((_L,), jnp.float32)
                        for c in range(_D // _L):
                            hv = hrows[i, pl.ds(oh + c * _L, _L)]
                            rv = rrows[i, pl.ds(orr + c * _L, _L)]
                            tv = trows[i, pl.ds(ot + c * _L, _L)]
                            acc = acc + jnp.abs(hv + rv - tv)
                        vec = jnp.where(lanes == j, jnp.sum(acc), vec)
                    obuf[pl.ds(chunk * _CH + g * _L, _L)] = vec
                    return carry

                lax.fori_loop(0, _CH // _L, group, 0)
            pltpu.sync_copy(obuf, out_hbm.at[pl.ds(base, _BW)])

        one_side(ph, pr, pt, pos_out)
        one_side(nh, nr, nt, neg_out)

    return trans_e


_KERNEL = _make_kernel()


@jax.jit
def kernel(pos_triples, neg_triples, ent_embs, rel_embs):
    pos = pos_triples.astype(jnp.int32)
    neg = neg_triples.astype(jnp.int32)
    ph, pr, pt = pos[:, 0], pos[:, 1], pos[:, 2]
    nh, nr, nt = neg[:, 0], neg[:, 1], neg[:, 2]
    ent2 = ent_embs.reshape(ent_embs.shape[0] // 2, 2 * _D)
    rel2 = rel_embs.reshape(rel_embs.shape[0] // 2, 2 * _D)
    pos_dist, neg_dist = _KERNEL(ph, pr, pt, nh, nr, nt, ent2, rel2)
    return pos_dist, neg_dist


# speedup vs baseline: 1.7138x; 1.7138x over previous
import functools

import jax
import jax.numpy as jnp
from jax import lax
from jax.experimental import pallas as pl
from jax.experimental.pallas import tpu as pltpu
from jax.experimental.pallas import tpu_sc as plsc
from jax.experimental.layout import Format, Layout, with_layout_constraint

_B = 16384
_D = 64
_NC = 2
_NS = 16
_NW = _NC * _NS
_BW = _B // _NW
_L = 16


def _make_kernel():
    mesh = plsc.VectorSubcoreMesh(core_axis_name="c", subcore_axis_name="s")

    @functools.partial(
        pl.kernel,
        mesh=mesh,
        compiler_params=pltpu.CompilerParams(
            needs_layout_passes=False, use_tc_tiling_on_sc=False),
        out_type=[
            jax.ShapeDtypeStruct((_B,), jnp.float32),
            jax.ShapeDtypeStruct((_B,), jnp.float32),
        ],
        scratch_types=[
            pltpu.VMEM((_BW,), jnp.int32),
            pltpu.VMEM((_BW,), jnp.int32),
            pltpu.VMEM((_BW,), jnp.int32),
            pltpu.VMEM((_BW, _D), jnp.float32),
            pltpu.VMEM((_BW, _D), jnp.float32),
            pltpu.VMEM((_BW, _D), jnp.float32),
            pltpu.VMEM((_BW,), jnp.float32),
            pltpu.SemaphoreType.DMA,
            pltpu.SemaphoreType.DMA,
            pltpu.SemaphoreType.DMA,
        ],
    )
    def trans_e(ph, pr, pt, nh, nr, nt, ent, rel, pos_out, neg_out,
                idx_h, idx_r, idx_t, hrows, rrows, trows, obuf,
                sem_h, sem_r, sem_t):
        wid = lax.axis_index("s") * _NC + lax.axis_index("c")
        base = wid * _BW
        lanes = lax.iota(jnp.int32, _L)

        def one_side(h_hbm, r_hbm, t_hbm, out_hbm):
            pltpu.sync_copy(h_hbm.at[pl.ds(base, _BW)], idx_h)
            pltpu.sync_copy(r_hbm.at[pl.ds(base, _BW)], idx_r)
            pltpu.sync_copy(t_hbm.at[pl.ds(base, _BW)], idx_t)
            ch = pltpu.async_copy(ent.at[idx_h], hrows, sem_h)
            cr = pltpu.async_copy(rel.at[idx_r], rrows, sem_r)
            ct = pltpu.async_copy(ent.at[idx_t], trows, sem_t)
            ch.wait()
            cr.wait()
            ct.wait()

            def group(g, carry):
                vec = jnp.zeros((_L,), jnp.float32)
                for j in range(_L):
                    i = g * _L + j
                    acc = jnp.zeros((_L,), jnp.float32)
                    for c in range(_D // _L):
                        hv = hrows[i, pl.ds(c * _L, _L)]
                        rv = rrows[i, pl.ds(c * _L, _L)]
                        tv = trows[i, pl.ds(c * _L, _L)]
                        acc = acc + jnp.abs(hv + rv - tv)
                    vec = jnp.where(lanes == j, jnp.sum(acc), vec)
                obuf[pl.ds(g * _L, _L)] = vec
                return carry

            lax.fori_loop(0, _BW // _L, group, 0)
            pltpu.sync_copy(obuf, out_hbm.at[pl.ds(base, _BW)])

        one_side(ph, pr, pt, pos_out)
        one_side(nh, nr, nt, neg_out)

    return trans_e


_KERNEL = _make_kernel()

@functools.lru_cache(maxsize=1)
def _row_major_linear():
    return Layout(major_to_minor=(0, 1), tiling=((16,),))


@jax.jit
def kernel(pos_triples, neg_triples, ent_embs, rel_embs):
    pos = pos_triples.astype(jnp.int32)
    neg = neg_triples.astype(jnp.int32)
    ph, pr, pt = pos[:, 0], pos[:, 1], pos[:, 2]
    nh, nr, nt = neg[:, 0], neg[:, 1], neg[:, 2]
    ent_lin = with_layout_constraint(ent_embs, _row_major_linear())
    rel_lin = with_layout_constraint(rel_embs, _row_major_linear())
    pos_dist, neg_dist = _KERNEL(ph, pr, pt, nh, nr, nt, ent_lin, rel_lin)
    return pos_dist, neg_dist
